# baseline (speedup 1.0000x reference)
import functools

import jax
import jax.numpy as jnp
from jax import lax
from jax.experimental import pallas as pl
from jax.experimental.pallas import tpu as pltpu
from jax.experimental.pallas import tpu_sc as plsc

N = 10000
E = 320000
D = 128
DW = 144
N_PAD = 10112
NC = 2
NS = 16
EW = 64
ROWS = 5120
E_PAD = ROWS * EW
ROWS_PER_TILE = ROWS // NS
CHUNK_ROWS = 8
NCHUNKS = ROWS_PER_TILE // CHUNK_ROWS
ZROWS = N_PAD // NS
NSLOT = 4

_mesh = plsc.VectorSubcoreMesh(
    core_axis_name="c", subcore_axis_name="s", num_cores=NC, num_subcores=NS)


@functools.partial(
    pl.kernel,
    out_type=jax.ShapeDtypeStruct((NC, N_PAD, DW), jnp.float32),
    mesh=_mesh,
    scratch_types=[
        pltpu.VMEM((2, CHUNK_ROWS, EW), jnp.int32),
        pltpu.VMEM((2, CHUNK_ROWS, EW), jnp.int32),
        pltpu.VMEM((NSLOT, EW, DW), jnp.float32),
        pltpu.VMEM_SHARED((N_PAD, DW), jnp.float32),
        pltpu.SemaphoreType.DMA,
        pltpu.SemaphoreType.DMA,
        pltpu.SemaphoreType.DMA,
    ],
    compiler_params=pltpu.CompilerParams(use_tc_tiling_on_sc=False),
)
def _sc_agg(idx_hbm, xp_hbm, zeros_hbm, out_hbm, gidx, sidx, rows, acc,
            gsem, ssem, isem):
    c = lax.axis_index("c")
    s = lax.axis_index("s")
    pltpu.sync_copy(zeros_hbm, acc.at[pl.ds(s * ZROWS, ZROWS)])
    plsc.subcore_barrier()

    base0 = s * ROWS_PER_TILE
    pltpu.sync_copy(idx_hbm.at[c, pl.ds(base0, CHUNK_ROWS)], gidx.at[0])
    pltpu.sync_copy(idx_hbm.at[1 - c, pl.ds(base0, CHUNK_ROWS)], sidx.at[0])
    pltpu.async_copy(xp_hbm.at[gidx.at[0, 0]], rows.at[0], gsem)
    pltpu.async_copy(xp_hbm.at[gidx.at[0, 1]], rows.at[1], gsem)

    def chunk_body(k, carry):
        slot = lax.rem(k, 2)
        nslot = 1 - slot
        nbase = s * ROWS_PER_TILE + (k + 1) * CHUNK_ROWS

        @pl.when(k + 1 < NCHUNKS)
        def _prefetch():
            pltpu.async_copy(idx_hbm.at[c, pl.ds(nbase, CHUNK_ROWS)],
                             gidx.at[nslot], isem)
            pltpu.async_copy(idx_hbm.at[1 - c, pl.ds(nbase, CHUNK_ROWS)],
                             sidx.at[nslot], isem)

        for j in range(CHUNK_ROWS):
            m = j % NSLOT
            pltpu.make_async_copy(
                xp_hbm.at[gidx.at[slot, j]], rows.at[m], gsem).wait()
            pltpu.async_copy(rows.at[m], acc.at[sidx.at[slot, j]], ssem,
                             add=True)
            f = (j + 2) % NSLOT
            if j >= 2:
                pltpu.make_async_copy(
                    rows.at[f], acc.at[sidx.at[slot, j]], ssem).wait()
            else:
                @pl.when(k > 0)
                def _retire():
                    pltpu.make_async_copy(
                        rows.at[f], acc.at[sidx.at[slot, j]], ssem).wait()
            if j + 2 < CHUNK_ROWS:
                pltpu.async_copy(xp_hbm.at[gidx.at[slot, j + 2]],
                                 rows.at[f], gsem)
            else:
                @pl.when(k + 1 < NCHUNKS)
                def _next_gather():
                    if j + 2 == CHUNK_ROWS:
                        pltpu.make_async_copy(
                            idx_hbm.at[c, pl.ds(nbase, CHUNK_ROWS)],
                            gidx.at[nslot], isem).wait()
                        pltpu.make_async_copy(
                            idx_hbm.at[1 - c, pl.ds(nbase, CHUNK_ROWS)],
                            sidx.at[nslot], isem).wait()
                    pltpu.async_copy(
                        xp_hbm.at[gidx.at[nslot, j + 2 - CHUNK_ROWS]],
                        rows.at[f], gsem)
        return carry

    lax.fori_loop(0, NCHUNKS, chunk_body, 0)
    pltpu.make_async_copy(
        rows.at[(CHUNK_ROWS - 2) % NSLOT], acc.at[sidx.at[0, 0]], ssem).wait()
    pltpu.make_async_copy(
        rows.at[(CHUNK_ROWS - 1) % NSLOT], acc.at[sidx.at[0, 0]], ssem).wait()
    plsc.subcore_barrier()
    pltpu.sync_copy(acc.at[pl.ds(s * ZROWS, ZROWS)],
                    out_hbm.at[c, pl.ds(s * ZROWS, ZROWS)])


_RB = 2528


def _elu(v):
    return jnp.where(v > 0, v, jnp.exp(jnp.minimum(v, 0.0)) - 1.0)


def _full(shape):
    return pl.BlockSpec(shape, lambda i: (0,) * len(shape))


def _agg_body(acc_ref, ws2d_ref, bs2d_ref, wd2s_ref, bd2s_ref,
              xin_ref, xout_ref):
    f32 = jnp.float32
    ai = acc_ref[0]
    mi = ai[:, :D] / jnp.maximum(ai[:, D:D + 1], 1.0)
    xin_ref[...] = _elu(
        jnp.dot(mi, ws2d_ref[...], preferred_element_type=f32) + bs2d_ref[...])
    ao = acc_ref[1]
    mo = ao[:, :D] / jnp.maximum(ao[:, D:D + 1], 1.0)
    xout_ref[...] = _elu(
        jnp.dot(mo, wd2s_ref[...], preferred_element_type=f32) + bd2s_ref[...])


def _dense_agg(acc, W_s2d, b_s2d, W_d2s, b_d2s):
    return pl.pallas_call(
        _agg_body,
        grid=(N_PAD // _RB,),
        in_specs=[
            pl.BlockSpec((NC, _RB, DW), lambda i: (0, i, 0)),
            _full((D, D)), _full((1, D)), _full((D, D)), _full((1, D)),
        ],
        out_specs=[pl.BlockSpec((_RB, D), lambda i: (i, 0))] * 2,
        out_shape=[jax.ShapeDtypeStruct((N, D), jnp.float32)] * 2,
    )(acc, W_s2d, b_s2d, W_d2s, b_d2s)


def _mlp_body(xp_ref, w1_ref, b1_ref, w2_ref, b2_ref, xself_ref):
    f32 = jnp.float32
    xv = xp_ref[:, :D]
    h = _elu(jnp.dot(xv, w1_ref[...], preferred_element_type=f32) + b1_ref[...])
    xself_ref[...] = _elu(
        jnp.dot(h, w2_ref[...], preferred_element_type=f32) + b2_ref[...])


def _dense_mlp(xp, W1, b1, W2, b2):
    return pl.pallas_call(
        _mlp_body,
        grid=(N_PAD // _RB,),
        in_specs=[
            pl.BlockSpec((_RB, DW), lambda i: (i, 0)),
            _full((D, 4 * D)), _full((1, 4 * D)), _full((4 * D, D)), _full((1, D)),
        ],
        out_specs=pl.BlockSpec((_RB, D), lambda i: (i, 0)),
        out_shape=jax.ShapeDtypeStruct((N, D), jnp.float32),
    )(xp, W1, b1, W2, b2)


def kernel(x, edge_index, W_s2d, b_s2d, W_d2s, b_d2s, W1, b1, W2, b2):
    src = edge_index[0].astype(jnp.int32)
    dst = edge_index[1].astype(jnp.int32)
    pad = jnp.full((E_PAD - E,), N, jnp.int32)
    idx = jnp.stack([
        jnp.concatenate([src, pad]).reshape(ROWS, EW),
        jnp.concatenate([dst, pad]).reshape(ROWS, EW),
    ])
    xp = jnp.zeros((N_PAD, DW), jnp.float32)
    xp = xp.at[:N, :D].set(x)
    xp = xp.at[:N, D].set(1.0)
    zeros = jnp.zeros((ZROWS, DW), jnp.float32)
    acc = _sc_agg(idx, xp, zeros)
    x_self = _dense_mlp(xp, W1, b1.reshape(1, 4 * D), W2, b2.reshape(1, D))
    x_in, x_out = _dense_agg(
        acc, W_s2d, b_s2d.reshape(1, D), W_d2s, b_d2s.reshape(1, D))
    return (x_in, x_out, x_self)


# speedup vs baseline: 1.3587x; 1.3587x over previous
import functools

import numpy as np

import jax
import jax.numpy as jnp
from jax import lax
from jax.experimental import pallas as pl
from jax.experimental.pallas import tpu as pltpu
from jax.experimental.pallas import tpu_sc as plsc

N = 10000
E = 320000
D = 128
N_PAD = 10112
NC = 2
NS = 16
EW = 64
ROWS = 5120
E_PAD = ROWS * EW
ROWS_PER_TILE = ROWS // NS
CHUNK_ROWS = 8
NCHUNKS = ROWS_PER_TILE // CHUNK_ROWS
ZROWS = N_PAD // NS
CW = 16

_QPERM = np.empty((D,), dtype=np.int32)
for _g in range(D // 32):
    for _i in range(16):
        _QPERM[32 * _g + 2 * _i] = 32 * _g + _i
        _QPERM[32 * _g + 2 * _i + 1] = 32 * _g + 16 + _i

_mesh = plsc.VectorSubcoreMesh(
    core_axis_name="c", subcore_axis_name="s", num_cores=NC, num_subcores=NS)


@functools.partial(
    pl.kernel,
    out_type=[jax.ShapeDtypeStruct((NC, N_PAD, D), jnp.float32),
              jax.ShapeDtypeStruct((NC, N_PAD, CW), jnp.float32)],
    mesh=_mesh,
    scratch_types=[
        pltpu.VMEM((2, CHUNK_ROWS, EW), jnp.int32),
        pltpu.VMEM((2, CHUNK_ROWS, EW), jnp.int32),
        pltpu.VMEM((2, EW, D), jnp.bfloat16),
        pltpu.VMEM((2, EW, D), jnp.float32),
        pltpu.VMEM((EW, CW), jnp.float32),
        pltpu.VMEM_SHARED((N_PAD, D), jnp.float32),
        pltpu.VMEM_SHARED((N_PAD, CW), jnp.float32),
        pltpu.SemaphoreType.DMA,
        pltpu.SemaphoreType.DMA,
        pltpu.SemaphoreType.DMA,
    ],
    compiler_params=pltpu.CompilerParams(use_tc_tiling_on_sc=False,
                                         needs_layout_passes=False),
)
def _sc_agg(idx_hbm, xb_hbm, zeros_hbm, zeros_cnt_hbm, out_hbm, cnt_hbm,
            gidx, sidx, rbf, rf32, ones, acc, acc_cnt, gsem, ssem, isem):
    c = lax.axis_index("c")
    s = lax.axis_index("s")
    pltpu.sync_copy(zeros_hbm, acc.at[pl.ds(s * ZROWS, ZROWS)])
    pltpu.sync_copy(zeros_cnt_hbm, acc_cnt.at[pl.ds(s * ZROWS, ZROWS)])

    def ones_body(r, carry):
        ones[r, :] = jnp.full((CW,), 1.0, jnp.float32)
        return carry

    lax.fori_loop(0, EW, ones_body, 0)
    plsc.subcore_barrier()

    base0 = s * ROWS_PER_TILE
    pltpu.sync_copy(idx_hbm.at[c, pl.ds(base0, CHUNK_ROWS)], gidx.at[0])
    pltpu.sync_copy(idx_hbm.at[1 - c, pl.ds(base0, CHUNK_ROWS)], sidx.at[0])
    pltpu.async_copy(xb_hbm.at[gidx.at[0, 0]], rbf.at[0], gsem)
    pltpu.async_copy(xb_hbm.at[gidx.at[0, 1]], rbf.at[1], gsem)

    def chunk_body(k, carry):
        slot = lax.rem(k, 2)
        nslot = 1 - slot
        nbase = s * ROWS_PER_TILE + (k + 1) * CHUNK_ROWS

        @pl.when(k + 1 < NCHUNKS)
        def _prefetch():
            pltpu.async_copy(idx_hbm.at[c, pl.ds(nbase, CHUNK_ROWS)],
                             gidx.at[nslot], isem)
            pltpu.async_copy(idx_hbm.at[1 - c, pl.ds(nbase, CHUNK_ROWS)],
                             sidx.at[nslot], isem)

        for j in range(CHUNK_ROWS):
            b = j % 2
            pltpu.make_async_copy(
                xb_hbm.at[gidx.at[slot, j]], rbf.at[b], gsem).wait()
            if j >= 2:
                pltpu.make_async_copy(
                    rf32.at[b], acc.at[sidx.at[slot, j]], ssem).wait()
                pltpu.make_async_copy(
                    ones, acc_cnt.at[sidx.at[slot, j]], ssem).wait()
            else:
                @pl.when(k > 0)
                def _retire():
                    pltpu.make_async_copy(
                        rf32.at[b], acc.at[sidx.at[slot, j]], ssem).wait()
                    pltpu.make_async_copy(
                        ones, acc_cnt.at[sidx.at[slot, j]], ssem).wait()

            def conv_body(r, carry):
                for g in range(D // 32):
                    ab = rbf[b, r, pl.ds(32 * g, 32)]
                    lo, hi = plsc.unpack(ab, format=plsc.PackFormat.INTERLEAVED)
                    rf32[b, r, pl.ds(32 * g, 16)] = lo
                    rf32[b, r, pl.ds(32 * g + 16, 16)] = hi
                return carry

            lax.fori_loop(0, EW, conv_body, 0)

            pltpu.async_copy(rf32.at[b], acc.at[sidx.at[slot, j]], ssem,
                             add=True)
            pltpu.async_copy(ones, acc_cnt.at[sidx.at[slot, j]], ssem,
                             add=True)
            if j + 2 < CHUNK_ROWS:
                pltpu.async_copy(xb_hbm.at[gidx.at[slot, j + 2]],
                                 rbf.at[b], gsem)
            else:
                @pl.when(k + 1 < NCHUNKS)
                def _next_gather():
                    if j + 2 == CHUNK_ROWS:
                        pltpu.make_async_copy(
                            idx_hbm.at[c, pl.ds(nbase, CHUNK_ROWS)],
                            gidx.at[nslot], isem).wait()
                        pltpu.make_async_copy(
                            idx_hbm.at[1 - c, pl.ds(nbase, CHUNK_ROWS)],
                            sidx.at[nslot], isem).wait()
                    pltpu.async_copy(
                        xb_hbm.at[gidx.at[nslot, j + 2 - CHUNK_ROWS]],
                        rbf.at[b], gsem)
        return carry

    lax.fori_loop(0, NCHUNKS, chunk_body, 0)
    for b in (0, 1):
        pltpu.make_async_copy(rf32.at[b], acc.at[sidx.at[0, 0]], ssem).wait()
        pltpu.make_async_copy(ones, acc_cnt.at[sidx.at[0, 0]], ssem).wait()
    plsc.subcore_barrier()
    pltpu.sync_copy(acc.at[pl.ds(s * ZROWS, ZROWS)],
                    out_hbm.at[c, pl.ds(s * ZROWS, ZROWS)])
    pltpu.sync_copy(acc_cnt.at[pl.ds(s * ZROWS, ZROWS)],
                    cnt_hbm.at[c, pl.ds(s * ZROWS, ZROWS)])


_RB = 2528


def _elu(v):
    return jnp.where(v > 0, v, jnp.exp(jnp.minimum(v, 0.0)) - 1.0)


def _full(shape):
    return pl.BlockSpec(shape, lambda i: (0,) * len(shape))


def _agg_body(acc_ref, cnt_ref, ws2d_ref, bs2d_ref, wd2s_ref, bd2s_ref,
              xin_ref, xout_ref):
    f32 = jnp.float32
    mi = acc_ref[0] / jnp.maximum(cnt_ref[0, :, 0:1], 1.0)
    xin_ref[...] = _elu(
        jnp.dot(mi, ws2d_ref[...], preferred_element_type=f32) + bs2d_ref[...])
    mo = acc_ref[1] / jnp.maximum(cnt_ref[1, :, 0:1], 1.0)
    xout_ref[...] = _elu(
        jnp.dot(mo, wd2s_ref[...], preferred_element_type=f32) + bd2s_ref[...])


def _dense_agg(acc, cnt, W_s2d, b_s2d, W_d2s, b_d2s):
    return pl.pallas_call(
        _agg_body,
        grid=(N_PAD // _RB,),
        in_specs=[
            pl.BlockSpec((NC, _RB, D), lambda i: (0, i, 0)),
            pl.BlockSpec((NC, _RB, CW), lambda i: (0, i, 0)),
            _full((D, D)), _full((1, D)), _full((D, D)), _full((1, D)),
        ],
        out_specs=[pl.BlockSpec((_RB, D), lambda i: (i, 0))] * 2,
        out_shape=[jax.ShapeDtypeStruct((N, D), jnp.float32)] * 2,
    )(acc, cnt, W_s2d, b_s2d, W_d2s, b_d2s)


def _mlp_body(x_ref, w1_ref, b1_ref, w2_ref, b2_ref, xself_ref):
    f32 = jnp.float32
    h = _elu(jnp.dot(x_ref[...], w1_ref[...], preferred_element_type=f32)
             + b1_ref[...])
    xself_ref[...] = _elu(
        jnp.dot(h, w2_ref[...], preferred_element_type=f32) + b2_ref[...])


def _dense_mlp(x, W1, b1, W2, b2):
    return pl.pallas_call(
        _mlp_body,
        grid=(N_PAD // _RB,),
        in_specs=[
            pl.BlockSpec((_RB, D), lambda i: (i, 0)),
            _full((D, 4 * D)), _full((1, 4 * D)), _full((4 * D, D)), _full((1, D)),
        ],
        out_specs=pl.BlockSpec((_RB, D), lambda i: (i, 0)),
        out_shape=jax.ShapeDtypeStruct((N, D), jnp.float32),
    )(x, W1, b1, W2, b2)


def kernel(x, edge_index, W_s2d, b_s2d, W_d2s, b_d2s, W1, b1, W2, b2):
    src = edge_index[0].astype(jnp.int32)
    dst = edge_index[1].astype(jnp.int32)
    pad = jnp.full((E_PAD - E,), N, jnp.int32)
    idx = jnp.stack([
        jnp.concatenate([src, pad]).reshape(ROWS, EW),
        jnp.concatenate([dst, pad]).reshape(ROWS, EW),
    ])
    xb = jnp.zeros((N_PAD, D), jnp.bfloat16)
    xb = xb.at[:N].set(x[:, _QPERM].astype(jnp.bfloat16))
    zeros = jnp.zeros((ZROWS, D), jnp.float32)
    zeros_cnt = jnp.zeros((ZROWS, CW), jnp.float32)
    acc, cnt = _sc_agg(idx, xb, zeros, zeros_cnt)
    x_self = _dense_mlp(x, W1, b1.reshape(1, 4 * D), W2, b2.reshape(1, D))
    x_in, x_out = _dense_agg(
        acc, cnt, W_s2d, b_s2d.reshape(1, D), W_d2s, b_d2s.reshape(1, D))
    return (x_in, x_out, x_self)


# speedup vs baseline: 1.3614x; 1.0020x over previous
import functools

import numpy as np

import jax
import jax.numpy as jnp
from jax import lax
from jax.experimental import pallas as pl
from jax.experimental.pallas import tpu as pltpu
from jax.experimental.pallas import tpu_sc as plsc

N = 10000
E = 320000
D = 128
N_PAD = 10112
NC = 2
NS = 16
EW = 64
ROWS = 5120
E_PAD = ROWS * EW
ROWS_PER_TILE = ROWS // NS
CHUNK_ROWS = 8
NCHUNKS = ROWS_PER_TILE // CHUNK_ROWS
ZROWS = N_PAD // NS
CW = 16

_QPERM = np.empty((D,), dtype=np.int32)
for _g in range(D // 32):
    for _i in range(16):
        _QPERM[32 * _g + 2 * _i] = 32 * _g + _i
        _QPERM[32 * _g + 2 * _i + 1] = 32 * _g + 16 + _i

_mesh = plsc.VectorSubcoreMesh(
    core_axis_name="c", subcore_axis_name="s", num_cores=NC, num_subcores=NS)


@functools.partial(
    pl.kernel,
    out_type=[jax.ShapeDtypeStruct((NC, N_PAD, D), jnp.float32),
              jax.ShapeDtypeStruct((NC, N_PAD, CW), jnp.float32)],
    mesh=_mesh,
    scratch_types=[
        pltpu.VMEM((2, CHUNK_ROWS, EW), jnp.int32),
        pltpu.VMEM((2, CHUNK_ROWS, EW), jnp.int32),
        pltpu.VMEM((4, EW, D), jnp.bfloat16),
        pltpu.VMEM((2, EW, D), jnp.float32),
        pltpu.VMEM((EW, CW), jnp.float32),
        pltpu.VMEM_SHARED((N_PAD, D), jnp.float32),
        pltpu.VMEM_SHARED((N_PAD, CW), jnp.float32),
        pltpu.SemaphoreType.DMA,
        pltpu.SemaphoreType.DMA,
        pltpu.SemaphoreType.DMA,
    ],
    compiler_params=pltpu.CompilerParams(use_tc_tiling_on_sc=False,
                                         needs_layout_passes=False),
)
def _sc_agg(idx_hbm, xb_hbm, zeros_hbm, zeros_cnt_hbm, out_hbm, cnt_hbm,
            gidx, sidx, rbf, rf32, ones, acc, acc_cnt, gsem, ssem, isem):
    c = lax.axis_index("c")
    s = lax.axis_index("s")
    pltpu.sync_copy(zeros_hbm, acc.at[pl.ds(s * ZROWS, ZROWS)])
    pltpu.sync_copy(zeros_cnt_hbm, acc_cnt.at[pl.ds(s * ZROWS, ZROWS)])

    def ones_body(r, carry):
        ones[r, :] = jnp.full((CW,), 1.0, jnp.float32)
        return carry

    lax.fori_loop(0, EW, ones_body, 0)
    plsc.subcore_barrier()

    base0 = s * ROWS_PER_TILE
    pltpu.sync_copy(idx_hbm.at[c, pl.ds(base0, CHUNK_ROWS)], gidx.at[0])
    pltpu.sync_copy(idx_hbm.at[1 - c, pl.ds(base0, CHUNK_ROWS)], sidx.at[0])
    pltpu.async_copy(xb_hbm.at[gidx.at[0, 0]], rbf.at[0], gsem)
    pltpu.async_copy(xb_hbm.at[gidx.at[0, 1]], rbf.at[1], gsem)

    def chunk_body(k, carry):
        slot = lax.rem(k, 2)
        nslot = 1 - slot
        nbase = s * ROWS_PER_TILE + (k + 1) * CHUNK_ROWS

        @pl.when(k + 1 < NCHUNKS)
        def _prefetch():
            pltpu.async_copy(idx_hbm.at[c, pl.ds(nbase, CHUNK_ROWS)],
                             gidx.at[nslot], isem)
            pltpu.async_copy(idx_hbm.at[1 - c, pl.ds(nbase, CHUNK_ROWS)],
                             sidx.at[nslot], isem)

        for j in range(CHUNK_ROWS):
            b4 = j % 4
            b = j % 2
            pltpu.make_async_copy(
                xb_hbm.at[gidx.at[slot, j]], rbf.at[b4], gsem).wait()
            f4 = (j + 2) % 4
            if j + 2 < CHUNK_ROWS:
                pltpu.async_copy(xb_hbm.at[gidx.at[slot, j + 2]],
                                 rbf.at[f4], gsem)
            else:
                @pl.when(k + 1 < NCHUNKS)
                def _next_gather():
                    if j + 2 == CHUNK_ROWS:
                        pltpu.make_async_copy(
                            idx_hbm.at[c, pl.ds(nbase, CHUNK_ROWS)],
                            gidx.at[nslot], isem).wait()
                        pltpu.make_async_copy(
                            idx_hbm.at[1 - c, pl.ds(nbase, CHUNK_ROWS)],
                            sidx.at[nslot], isem).wait()
                    pltpu.async_copy(
                        xb_hbm.at[gidx.at[nslot, j + 2 - CHUNK_ROWS]],
                        rbf.at[f4], gsem)
            if j >= 2:
                pltpu.make_async_copy(
                    rf32.at[b], acc.at[sidx.at[slot, j]], ssem).wait()
                pltpu.make_async_copy(
                    ones, acc_cnt.at[sidx.at[slot, j]], ssem).wait()
            else:
                @pl.when(k > 0)
                def _retire():
                    pltpu.make_async_copy(
                        rf32.at[b], acc.at[sidx.at[slot, j]], ssem).wait()
                    pltpu.make_async_copy(
                        ones, acc_cnt.at[sidx.at[slot, j]], ssem).wait()

            def conv_body(r, carry):
                for g in range(D // 32):
                    ab = rbf[b4, r, pl.ds(32 * g, 32)]
                    lo, hi = plsc.unpack(ab, format=plsc.PackFormat.INTERLEAVED)
                    rf32[b, r, pl.ds(32 * g, 16)] = lo
                    rf32[b, r, pl.ds(32 * g + 16, 16)] = hi
                return carry

            lax.fori_loop(0, EW, conv_body, 0)

            pltpu.async_copy(rf32.at[b], acc.at[sidx.at[slot, j]], ssem,
                             add=True)
            pltpu.async_copy(ones, acc_cnt.at[sidx.at[slot, j]], ssem,
                             add=True)
        return carry

    lax.fori_loop(0, NCHUNKS, chunk_body, 0)
    for b in (0, 1):
        pltpu.make_async_copy(rf32.at[b], acc.at[sidx.at[0, 0]], ssem).wait()
        pltpu.make_async_copy(ones, acc_cnt.at[sidx.at[0, 0]], ssem).wait()
    plsc.subcore_barrier()
    pltpu.sync_copy(acc.at[pl.ds(s * ZROWS, ZROWS)],
                    out_hbm.at[c, pl.ds(s * ZROWS, ZROWS)])
    pltpu.sync_copy(acc_cnt.at[pl.ds(s * ZROWS, ZROWS)],
                    cnt_hbm.at[c, pl.ds(s * ZROWS, ZROWS)])


_RB = 2528


def _elu(v):
    return jnp.where(v > 0, v, jnp.exp(jnp.minimum(v, 0.0)) - 1.0)


def _full(shape):
    return pl.BlockSpec(shape, lambda i: (0,) * len(shape))


def _agg_body(acc_ref, cnt_ref, ws2d_ref, bs2d_ref, wd2s_ref, bd2s_ref,
              xin_ref, xout_ref):
    f32 = jnp.float32
    mi = acc_ref[0] / jnp.maximum(cnt_ref[0, :, 0:1], 1.0)
    xin_ref[...] = _elu(
        jnp.dot(mi, ws2d_ref[...], preferred_element_type=f32) + bs2d_ref[...])
    mo = acc_ref[1] / jnp.maximum(cnt_ref[1, :, 0:1], 1.0)
    xout_ref[...] = _elu(
        jnp.dot(mo, wd2s_ref[...], preferred_element_type=f32) + bd2s_ref[...])


def _dense_agg(acc, cnt, W_s2d, b_s2d, W_d2s, b_d2s):
    return pl.pallas_call(
        _agg_body,
        grid=(N_PAD // _RB,),
        in_specs=[
            pl.BlockSpec((NC, _RB, D), lambda i: (0, i, 0)),
            pl.BlockSpec((NC, _RB, CW), lambda i: (0, i, 0)),
            _full((D, D)), _full((1, D)), _full((D, D)), _full((1, D)),
        ],
        out_specs=[pl.BlockSpec((_RB, D), lambda i: (i, 0))] * 2,
        out_shape=[jax.ShapeDtypeStruct((N, D), jnp.float32)] * 2,
    )(acc, cnt, W_s2d, b_s2d, W_d2s, b_d2s)


def _mlp_body(x_ref, w1_ref, b1_ref, w2_ref, b2_ref, xself_ref):
    f32 = jnp.float32
    h = _elu(jnp.dot(x_ref[...], w1_ref[...], preferred_element_type=f32)
             + b1_ref[...])
    xself_ref[...] = _elu(
        jnp.dot(h, w2_ref[...], preferred_element_type=f32) + b2_ref[...])


def _dense_mlp(x, W1, b1, W2, b2):
    return pl.pallas_call(
        _mlp_body,
        grid=(N_PAD // _RB,),
        in_specs=[
            pl.BlockSpec((_RB, D), lambda i: (i, 0)),
            _full((D, 4 * D)), _full((1, 4 * D)), _full((4 * D, D)), _full((1, D)),
        ],
        out_specs=pl.BlockSpec((_RB, D), lambda i: (i, 0)),
        out_shape=jax.ShapeDtypeStruct((N, D), jnp.float32),
    )(x, W1, b1, W2, b2)


def kernel(x, edge_index, W_s2d, b_s2d, W_d2s, b_d2s, W1, b1, W2, b2):
    src = edge_index[0].astype(jnp.int32)
    dst = edge_index[1].astype(jnp.int32)
    pad = jnp.full((E_PAD - E,), N, jnp.int32)
    idx = jnp.stack([
        jnp.concatenate([src, pad]).reshape(ROWS, EW),
        jnp.concatenate([dst, pad]).reshape(ROWS, EW),
    ])
    xb = jnp.zeros((N_PAD, D), jnp.bfloat16)
    xb = xb.at[:N].set(x[:, _QPERM].astype(jnp.bfloat16))
    zeros = jnp.zeros((ZROWS, D), jnp.float32)
    zeros_cnt = jnp.zeros((ZROWS, CW), jnp.float32)
    x_self = _dense_mlp(x, W1, b1.reshape(1, 4 * D), W2, b2.reshape(1, D))
    acc, cnt = _sc_agg(idx, xb, zeros, zeros_cnt)
    x_in, x_out = _dense_agg(
        acc, cnt, W_s2d, b_s2d.reshape(1, D), W_d2s, b_d2s.reshape(1, D))
    return (x_in, x_out, x_self)
